# baseline (speedup 1.0000x reference)
import functools

import jax
import jax.numpy as jnp
from jax import lax
from jax.experimental import pallas as pl
from jax.experimental.pallas import tpu as pltpu
from jax.experimental.pallas import tpu_sc as plsc

_NC = 2
_NS = 16
_NW = _NC * _NS

_CE = 512
_LANES = 16
_GROUPS = _CE // _LANES


def _sqrt_newton(s):
    i = lax.bitcast_convert_type(s, jnp.int32)
    i = jnp.int32(0x1FBD1DF5) + lax.shift_right_arithmetic(i, 1)
    y = lax.bitcast_convert_type(i, jnp.float32)
    y = 0.5 * (y + s / y)
    y = 0.5 * (y + s / y)
    return y


def _make_sc_kernel(N, chunks_per_worker):
    mesh = plsc.VectorSubcoreMesh(core_axis_name="c", subcore_axis_name="s")
    cw = chunks_per_worker * _CE

    @functools.partial(
        pl.kernel,
        out_type=jax.ShapeDtypeStruct((_NW * 32,), jnp.float32),
        mesh=mesh,
        compiler_params=pltpu.CompilerParams(needs_layout_passes=False,
                                             use_tc_tiling_on_sc=False),
        scratch_types=[
            pltpu.VMEM((8 + _CE,), jnp.int32),
            pltpu.VMEM((6, _CE), jnp.int32),
            pltpu.VMEM((12, _CE), jnp.float32),
            pltpu.VMEM((32,), jnp.float32),
            pltpu.SemaphoreType.DMA,
        ],
    )
    def sc_kernel(keys_hbm, v1_hbm, v2_hbm, out_hbm,
                  keys_v, idx_v, comp_v, out_v, sem):
        ci = lax.axis_index("c")
        si = lax.axis_index("s")
        wid = si * _NC + ci
        base = 8 + wid * cw
        zero = jnp.zeros((_LANES,), jnp.float32)

        def chunk_body(k, carry):
            acc_s, acc_c = carry
            off = base + k * _CE
            pltpu.sync_copy(keys_hbm.at[pl.ds(off - 8, 8 + _CE)], keys_v)
            for j in range(_GROUPS):
                kv = keys_v[pl.ds(8 + j * _LANES, _LANES)]
                lo = lax.shift_right_logical(kv, 16)
                hi = lax.bitwise_and(kv, jnp.int32(0xFFFF))
                lo = jnp.minimum(lo, jnp.int32(N - 1))
                hi = jnp.minimum(hi, jnp.int32(N - 1))
                l3 = lo * 3
                h3 = hi * 3
                s = pl.ds(j * _LANES, _LANES)
                idx_v[0, s] = l3
                idx_v[1, s] = l3 + 1
                idx_v[2, s] = l3 + 2
                idx_v[3, s] = h3
                idx_v[4, s] = h3 + 1
                idx_v[5, s] = h3 + 2
            copies = [
                pltpu.async_copy(v1_hbm.at[idx_v.at[t]], comp_v.at[t], sem)
                for t in range(6)
            ] + [
                pltpu.async_copy(v2_hbm.at[idx_v.at[t]], comp_v.at[6 + t], sem)
                for t in range(6)
            ]
            for c in copies:
                c.wait()
            for g in range(_GROUPS):
                s = pl.ds(g * _LANES, _LANES)
                dx1 = comp_v[0, s] - comp_v[3, s]
                dy1 = comp_v[1, s] - comp_v[4, s]
                dz1 = comp_v[2, s] - comp_v[5, s]
                dx2 = comp_v[6, s] - comp_v[9, s]
                dy2 = comp_v[7, s] - comp_v[10, s]
                dz2 = comp_v[8, s] - comp_v[11, s]
                s1 = dx1 * dx1 + dy1 * dy1 + dz1 * dz1 + jnp.float32(1e-12)
                s2 = dx2 * dx2 + dy2 * dy2 + dz2 * dz2 + jnp.float32(1e-12)
                d = _sqrt_newton(s1) - _sqrt_newton(s2)
                kv = keys_v[pl.ds(8 + g * _LANES, _LANES)]
                pv = keys_v[pl.ds(7 + g * _LANES, _LANES)]
                nondeg = lax.bitwise_and(
                    lax.bitwise_xor(lax.shift_right_logical(kv, 16), kv),
                    jnp.int32(0xFFFF)) != 0
                first = jnp.not_equal(kv, pv)
                w = jnp.where(jnp.logical_and(first, nondeg),
                              jnp.float32(1.0), jnp.float32(0.0))
                acc_s = acc_s + w * (d * d)
                acc_c = acc_c + w
            return acc_s, acc_c

        acc_s, acc_c = lax.fori_loop(0, chunks_per_worker, chunk_body,
                                     (zero, zero))
        out_v[pl.ds(0, 16)] = acc_s
        out_v[pl.ds(16, 16)] = acc_c
        pltpu.sync_copy(out_v, out_hbm.at[pl.ds(wid * 32, 32)])

    return sc_kernel


def kernel(vert1, vert2, face):
    B, N, _ = vert1.shape
    F = face.shape[1]
    E = 3 * F
    per_worker = -(-E // (_NW * _CE)) * _CE
    padded = per_worker * _NW
    chunks_per_worker = per_worker // _CE

    f = face.astype(jnp.int32)
    a, b_, c = f[..., 0], f[..., 1], f[..., 2]
    p = jnp.stack([a, b_, b_, c, c, a], axis=-1).reshape(B, E, 2)
    lo = jnp.min(p, axis=-1).astype(jnp.uint32)
    hi = jnp.max(p, axis=-1).astype(jnp.uint32)
    sent_head = jnp.zeros((B, 8), jnp.uint32)
    sent_tail = jnp.full((B, padded - E), 0xFFFFFFFF, jnp.uint32)
    keys = jnp.concatenate([sent_head, (lo << 16) | hi, sent_tail], axis=1)

    sc = _make_sc_kernel(N, chunks_per_worker)
    losses = []
    for b in range(B):
        kb = lax.sort(keys[b], dimension=0, is_stable=False)
        kb_i32 = lax.bitcast_convert_type(kb, jnp.int32)
        parts = sc(kb_i32, vert1[b].reshape(-1), vert2[b].reshape(-1))
        parts = parts.reshape(_NW, 2, 16)
        losses.append(jnp.sum(parts[:, 0, :]) / jnp.sum(parts[:, 1, :]))
    return jnp.mean(jnp.stack(losses))


# speedup vs baseline: 1.0685x; 1.0685x over previous
import functools

import jax
import jax.numpy as jnp
from jax import lax
from jax.experimental import pallas as pl
from jax.experimental.pallas import tpu as pltpu
from jax.experimental.pallas import tpu_sc as plsc

_NC = 2
_NS = 16
_NW = _NC * _NS

_CE = 512
_LANES = 16
_GROUPS = _CE // _LANES


def _sqrt_newton(s):
    i = lax.bitcast_convert_type(s, jnp.int32)
    i = jnp.int32(0x1FBD1DF5) + lax.shift_right_arithmetic(i, 1)
    y = lax.bitcast_convert_type(i, jnp.float32)
    y = 0.5 * (y + s / y)
    y = 0.5 * (y + s / y)
    return y


def _make_sc_kernel(B, N, chunks_per_worker):
    mesh = plsc.VectorSubcoreMesh(core_axis_name="c", subcore_axis_name="s")
    cw = chunks_per_worker * _CE
    wpb = _NW // B

    @functools.partial(
        pl.kernel,
        out_type=jax.ShapeDtypeStruct((_NW * 32,), jnp.float32),
        mesh=mesh,
        compiler_params=pltpu.CompilerParams(needs_layout_passes=False,
                                             use_tc_tiling_on_sc=False),
        scratch_types=[
            pltpu.VMEM((8 + _CE,), jnp.int32),
            pltpu.VMEM((6, _CE), jnp.int32),
            pltpu.VMEM((12, _CE), jnp.float32),
            pltpu.VMEM((32,), jnp.float32),
            pltpu.SemaphoreType.DMA,
        ],
    )
    def sc_kernel(keys_hbm, v1_hbm, v2_hbm, out_hbm,
                  keys_v, idx_v, comp_v, out_v, sem):
        ci = lax.axis_index("c")
        si = lax.axis_index("s")
        wid = si * _NC + ci
        b = wid // wpb
        base = 8 + (wid % wpb) * cw
        tb3 = b * (3 * N)
        zero = jnp.zeros((_LANES,), jnp.float32)

        def chunk_body(k, carry):
            acc_s, acc_c = carry
            off = base + k * _CE
            pltpu.sync_copy(keys_hbm.at[b, pl.ds(off - 8, 8 + _CE)], keys_v)
            for j in range(_GROUPS):
                kv = keys_v[pl.ds(8 + j * _LANES, _LANES)]
                lo = lax.shift_right_logical(kv, 16)
                hi = lax.bitwise_and(kv, jnp.int32(0xFFFF))
                lo = jnp.minimum(lo, jnp.int32(N - 1))
                hi = jnp.minimum(hi, jnp.int32(N - 1))
                l3 = tb3 + lo * 3
                h3 = tb3 + hi * 3
                s = pl.ds(j * _LANES, _LANES)
                idx_v[0, s] = l3
                idx_v[1, s] = l3 + 1
                idx_v[2, s] = l3 + 2
                idx_v[3, s] = h3
                idx_v[4, s] = h3 + 1
                idx_v[5, s] = h3 + 2
            copies = [
                pltpu.async_copy(v1_hbm.at[idx_v.at[t]], comp_v.at[t], sem)
                for t in range(6)
            ] + [
                pltpu.async_copy(v2_hbm.at[idx_v.at[t]], comp_v.at[6 + t], sem)
                for t in range(6)
            ]
            for c in copies:
                c.wait()
            for g in range(_GROUPS):
                s = pl.ds(g * _LANES, _LANES)
                dx1 = comp_v[0, s] - comp_v[3, s]
                dy1 = comp_v[1, s] - comp_v[4, s]
                dz1 = comp_v[2, s] - comp_v[5, s]
                dx2 = comp_v[6, s] - comp_v[9, s]
                dy2 = comp_v[7, s] - comp_v[10, s]
                dz2 = comp_v[8, s] - comp_v[11, s]
                s1 = dx1 * dx1 + dy1 * dy1 + dz1 * dz1 + jnp.float32(1e-12)
                s2 = dx2 * dx2 + dy2 * dy2 + dz2 * dz2 + jnp.float32(1e-12)
                d = _sqrt_newton(s1) - _sqrt_newton(s2)
                kv = keys_v[pl.ds(8 + g * _LANES, _LANES)]
                pv = keys_v[pl.ds(7 + g * _LANES, _LANES)]
                nondeg = lax.bitwise_and(
                    lax.bitwise_xor(lax.shift_right_logical(kv, 16), kv),
                    jnp.int32(0xFFFF)) != 0
                first = jnp.not_equal(kv, pv)
                w = jnp.where(jnp.logical_and(first, nondeg),
                              jnp.float32(1.0), jnp.float32(0.0))
                acc_s = acc_s + w * (d * d)
                acc_c = acc_c + w
            return acc_s, acc_c

        acc_s, acc_c = lax.fori_loop(0, chunks_per_worker, chunk_body,
                                     (zero, zero))
        out_v[pl.ds(0, 16)] = acc_s
        out_v[pl.ds(16, 16)] = acc_c
        pltpu.sync_copy(out_v, out_hbm.at[pl.ds(wid * 32, 32)])

    return sc_kernel


def kernel(vert1, vert2, face):
    B, N, _ = vert1.shape
    F = face.shape[1]
    E = 3 * F
    wpb = _NW // B
    per_worker = -(-E // (wpb * _CE)) * _CE
    padded = per_worker * wpb
    chunks_per_worker = per_worker // _CE

    f = face.astype(jnp.int32)
    a, b_, c = f[..., 0], f[..., 1], f[..., 2]
    p = jnp.stack([a, b_, b_, c, c, a], axis=-1).reshape(B, E, 2)
    lo = jnp.min(p, axis=-1).astype(jnp.uint32)
    hi = jnp.max(p, axis=-1).astype(jnp.uint32)
    sent_head = jnp.zeros((B, 8), jnp.uint32)
    sent_tail = jnp.full((B, padded - E), 0xFFFFFFFF, jnp.uint32)
    keys = jnp.concatenate([sent_head, (lo << 16) | hi, sent_tail], axis=1)
    keys = lax.sort(keys, dimension=1, is_stable=False)
    keys_i32 = lax.bitcast_convert_type(keys, jnp.int32)

    sc = _make_sc_kernel(B, N, chunks_per_worker)
    parts = sc(keys_i32, vert1.reshape(-1), vert2.reshape(-1))

    parts = parts.reshape(B, wpb, 2, 16)
    sums = jnp.sum(parts[:, :, 0, :], axis=(1, 2))
    cnts = jnp.sum(parts[:, :, 1, :], axis=(1, 2))
    return jnp.mean(sums / cnts)
